# baseline (speedup 1.0000x reference)
import functools

import jax
import jax.numpy as jnp
from jax import lax
from jax.experimental import pallas as pl
from jax.experimental.pallas import tpu as pltpu
from jax.experimental.pallas import tpu_sc as plsc

_HIDDEN = 1152
_BATCH = 16384
_NUM_CORES = 2
_NUM_SUBCORES = 16
_NW = _NUM_CORES * _NUM_SUBCORES
_S = 1536
_ROWS_PER_W = _S // _NW
_REP = 4
_N_OUT = _ROWS_PER_W // _REP

_TC_ROWS = _BATCH - _S
_TC_BLOCK = 512
assert _TC_ROWS % _TC_BLOCK == 0


@functools.partial(
    pl.kernel,
    out_type=jax.ShapeDtypeStruct((_S, _HIDDEN), jnp.float32),
    mesh=plsc.VectorSubcoreMesh(core_axis_name="c", subcore_axis_name="s"),
    scratch_types=[
        pltpu.VMEM((_REP, _HIDDEN), jnp.float32),
        pltpu.SemaphoreType.DMA,
    ],
)
def _sc_broadcast(table_hbm, out_hbm, buf, sem):
    wid = lax.axis_index("s") * _NUM_CORES + lax.axis_index("c")
    pltpu.sync_copy(table_hbm.at[0], buf.at[0])

    def _fill_row(r, carry):
        for c in range(_HIDDEN // 16):
            buf[r, pl.ds(c * 16, 16)] = buf[0, pl.ds(c * 16, 16)]
        return carry

    lax.fori_loop(1, _REP, _fill_row, 0)
    base = wid * _ROWS_PER_W
    copies = [
        pltpu.async_copy(buf, out_hbm.at[pl.ds(base + i * _REP, _REP)], sem)
        for i in range(_N_OUT)
    ]
    for c in copies:
        c.wait()


def _tc_body(table_ref, out_ref):
    out_ref[...] = jnp.broadcast_to(table_ref[...], (_TC_BLOCK, _HIDDEN))


_tc_broadcast = pl.pallas_call(
    _tc_body,
    out_shape=jax.ShapeDtypeStruct((_TC_ROWS, _HIDDEN), jnp.float32),
    grid=(_TC_ROWS // _TC_BLOCK,),
    in_specs=[pl.BlockSpec((1, _HIDDEN), lambda i: (0, 0))],
    out_specs=pl.BlockSpec((_TC_BLOCK, _HIDDEN), lambda i: (i, 0)),
)


def kernel(labels, table):
    del labels
    sc_part = _sc_broadcast(table)
    tc_part = _tc_broadcast(table)
    return jnp.concatenate([sc_part, tc_part], axis=0)


# speedup vs baseline: 1.1324x; 1.1324x over previous
import functools

import jax
import jax.numpy as jnp
from jax import lax
from jax.experimental import pallas as pl
from jax.experimental.pallas import tpu as pltpu
from jax.experimental.pallas import tpu_sc as plsc

_HIDDEN = 1152
_BATCH = 16384
_NUM_CORES = 2
_ROWS_PER_C = _BATCH // _NUM_CORES
_REP = 128
_N_OUT = _ROWS_PER_C // _REP


@functools.partial(
    pl.kernel,
    out_type=jax.ShapeDtypeStruct((_BATCH, _HIDDEN), jnp.float32),
    mesh=plsc.ScalarSubcoreMesh(axis_name="c"),
    scratch_types=[
        pltpu.MemorySpace.VMEM_SHARED((_REP, _HIDDEN), jnp.float32),
        pltpu.SemaphoreType.DMA,
    ],
)
def _scs_broadcast(table_hbm, out_hbm, buf, sem):
    cid = lax.axis_index("c")
    fills = [
        pltpu.async_copy(table_hbm.at[0], buf.at[i], sem) for i in range(_REP)
    ]
    for c in fills:
        c.wait()
    base = cid * _ROWS_PER_C
    copies = [
        pltpu.async_copy(buf, out_hbm.at[pl.ds(base + i * _REP, _REP)], sem)
        for i in range(_N_OUT)
    ]
    for c in copies:
        c.wait()


def kernel(labels, table):
    del labels
    return _scs_broadcast(table)


# speedup vs baseline: 1.9476x; 1.7199x over previous
import functools

import jax
import jax.numpy as jnp
from jax import lax
from jax.experimental import pallas as pl
from jax.experimental.pallas import tpu as pltpu
from jax.experimental.pallas import tpu_sc as plsc

_HIDDEN = 1152
_BATCH = 16384
_NUM_CORES = 2
_NUM_SUBCORES = 16
_NW = _NUM_CORES * _NUM_SUBCORES
_ROWS_PER_W = _BATCH // _NW
_REP = 4
_N_OUT = _ROWS_PER_W // _REP


@functools.partial(
    pl.kernel,
    out_type=jax.ShapeDtypeStruct((_BATCH, _HIDDEN), jnp.float32),
    mesh=plsc.VectorSubcoreMesh(core_axis_name="c", subcore_axis_name="s"),
    scratch_types=[
        pltpu.VMEM((_REP, _HIDDEN), jnp.float32),
        pltpu.SemaphoreType.DMA,
    ],
)
def _broadcast_row(table_hbm, out_hbm, buf, sem):
    wid = lax.axis_index("s") * _NUM_CORES + lax.axis_index("c")
    pltpu.sync_copy(table_hbm.at[0], buf.at[0])

    def _fill_row(r, carry):
        for c in range(_HIDDEN // 16):
            buf[r, pl.ds(c * 16, 16)] = buf[0, pl.ds(c * 16, 16)]
        return carry

    lax.fori_loop(1, _REP, _fill_row, 0)
    base = wid * _ROWS_PER_W
    copies = [
        pltpu.async_copy(buf, out_hbm.at[pl.ds(base + i * _REP, _REP)], sem)
        for i in range(_N_OUT)
    ]
    for c in copies:
        c.wait()


def kernel(labels, table):
    del labels
    return _broadcast_row(table)
